# Initial kernel scaffold; baseline (speedup 1.0000x reference)
import jax
import jax.numpy as jnp
from jax.experimental import pallas as pl


def kernel(x, edge_index, batch, W1, b1, W2, b2, W3, b3):
    raise NotImplementedError("write your pallas kernel here")


# speedup vs baseline: 7.4402x; 7.4402x over previous
import functools

import jax
import jax.numpy as jnp
from jax import lax
from jax.experimental import pallas as pl
from jax.experimental.pallas import tpu as pltpu
from jax.experimental.pallas import tpu_sc as plsc

N_NODES = 10000
N_EDGES = 160000
D_IN = 256
D_HID = 256
D_OUT = 128
N_GRAPHS = 128

NC = 2
NS = 16

ROW_STRIDE = 624
ROW_COPY = 640
EDGES_PER_TILE = N_EDGES // NS
EDGES_PER_TILE_DEG = N_EDGES // (NC * NS)
CH = 80
CH_DEG = 40

R_BLK = 1000
N_RBLK = N_NODES // R_BLK


def _deg_kernel(dst_hbm, ones_hbm, zeros_hbm, out_hbm, dst_v, ones_v, acc, sem):
    c = lax.axis_index("c")
    s = lax.axis_index("s")
    r0 = s * ROW_STRIDE
    pltpu.sync_copy(ones_hbm, ones_v)
    pltpu.sync_copy(zeros_hbm, acc.at[pl.ds(r0, ROW_COPY)])
    plsc.subcore_barrier()

    e0 = (c * NS + s) * EDGES_PER_TILE_DEG

    def body(i, carry):
        base = e0 + i * CH_DEG
        pltpu.sync_copy(dst_hbm.at[pl.ds(base, CH_DEG)], dst_v)
        pltpu.sync_copy(ones_v, acc.at[dst_v], add=True)
        return carry

    lax.fori_loop(0, EDGES_PER_TILE_DEG // CH_DEG, body, 0)
    plsc.subcore_barrier()
    pltpu.sync_copy(
        acc.at[pl.ds(r0, ROW_COPY)],
        out_hbm.at[pl.ds(c * N_NODES + r0, ROW_COPY)],
    )


def _sc_degree(dst):
    ones = jnp.ones((CH_DEG, 128), jnp.float32)
    zeros = jnp.zeros((ROW_COPY, 128), jnp.float32)
    mesh = plsc.VectorSubcoreMesh(core_axis_name="c", subcore_axis_name="s")
    k = functools.partial(
        pl.kernel,
        mesh=mesh,
        out_type=jax.ShapeDtypeStruct((NC * N_NODES, 128), jnp.float32),
        scratch_types=[
            pltpu.VMEM((CH_DEG,), jnp.int32),
            pltpu.VMEM((CH_DEG, 128), jnp.float32),
            pltpu.VMEM_SHARED((N_NODES, 128), jnp.float32),
            pltpu.SemaphoreType.DMA,
        ],
    )(_deg_kernel)
    return k(dst, ones, zeros)


def _edge_kernel(y_hbm, src2_hbm, dst_hbm, zeros_hbm, out_hbm,
                 src_v, dst_v, rows_v, acc, sem):
    c = lax.axis_index("c")
    s = lax.axis_index("s")
    r0 = s * ROW_STRIDE
    pltpu.sync_copy(zeros_hbm, acc.at[pl.ds(r0, ROW_COPY)])
    plsc.subcore_barrier()

    e0 = c * N_EDGES + s * EDGES_PER_TILE
    d0 = s * EDGES_PER_TILE

    def body(i, carry):
        pltpu.sync_copy(src2_hbm.at[pl.ds(e0 + i * CH, CH)], src_v)
        pltpu.sync_copy(dst_hbm.at[pl.ds(d0 + i * CH, CH)], dst_v)
        pltpu.async_copy(y_hbm.at[src_v], rows_v, sem).wait()
        pltpu.sync_copy(rows_v, acc.at[dst_v], add=True)
        return carry

    lax.fori_loop(0, EDGES_PER_TILE // CH, body, 0)
    plsc.subcore_barrier()
    pltpu.sync_copy(
        acc.at[pl.ds(r0, ROW_COPY)],
        out_hbm.at[pl.ds(c * N_NODES + r0, ROW_COPY)],
    )


def _sc_edge_pass(y_split, src2, dst):
    zeros = jnp.zeros((ROW_COPY, 128), jnp.float32)
    mesh = plsc.VectorSubcoreMesh(core_axis_name="c", subcore_axis_name="s")
    k = functools.partial(
        pl.kernel,
        mesh=mesh,
        out_type=jax.ShapeDtypeStruct((NC * N_NODES, 128), jnp.float32),
        scratch_types=[
            pltpu.VMEM((CH,), jnp.int32),
            pltpu.VMEM((CH,), jnp.int32),
            pltpu.VMEM((CH, 128), jnp.float32),
            pltpu.VMEM_SHARED((N_NODES, 128), jnp.float32),
            pltpu.SemaphoreType.DMA,
        ],
    )(_edge_kernel)
    return k(y_split, src2, dst, zeros)


def _mm1_body(x_ref, w_ref, p0_ref, p1_ref, out_ref):
    deg = p0_ref[:, 0:1] + p1_ref[:, 0:1] + 1.0
    dis = lax.rsqrt(deg)
    out_ref[...] = dis * jnp.dot(x_ref[...], w_ref[...],
                                 preferred_element_type=jnp.float32)


def _tc_mm1(x, w1, degp):
    return pl.pallas_call(
        _mm1_body,
        grid=(N_RBLK, 2),
        in_specs=[
            pl.BlockSpec((R_BLK, D_IN), lambda r, c: (r, 0)),
            pl.BlockSpec((D_IN, 128), lambda r, c: (0, c)),
            pl.BlockSpec((R_BLK, 128), lambda r, c: (r, 0)),
            pl.BlockSpec((R_BLK, 128), lambda r, c: (N_RBLK + r, 0)),
        ],
        out_specs=pl.BlockSpec((R_BLK, 128), lambda r, c: (c * N_RBLK + r, 0)),
        out_shape=jax.ShapeDtypeStruct((NC * N_NODES, 128), jnp.float32),
    )(x, w1, degp, degp)


def _mm2_body(a0_ref, a1_ref, y0_ref, y1_ref, b_ref, w_ref, p0_ref, p1_ref,
              out_ref):
    deg = p0_ref[:, 0:1] + p1_ref[:, 0:1] + 1.0
    dis = lax.rsqrt(deg)
    h_l = jnp.maximum(dis * (a0_ref[...] + y0_ref[...]) + b_ref[0:1, :], 0.0)
    h_r = jnp.maximum(dis * (a1_ref[...] + y1_ref[...]) + b_ref[1:2, :], 0.0)
    h = jnp.concatenate([h_l, h_r], axis=1)
    out_ref[...] = dis * jnp.dot(h, w_ref[...],
                                 preferred_element_type=jnp.float32)


def _tc_mm2(acc1, y1, b1_2d, w2, degp):
    half = pl.BlockSpec((R_BLK, 128), lambda r, c: (r, 0))
    half_hi = pl.BlockSpec((R_BLK, 128), lambda r, c: (N_RBLK + r, 0))
    return pl.pallas_call(
        _mm2_body,
        grid=(N_RBLK, 2),
        in_specs=[
            half, half_hi, half, half_hi,
            pl.BlockSpec((2, 128), lambda r, c: (0, 0)),
            pl.BlockSpec((D_HID, 128), lambda r, c: (0, c)),
            pl.BlockSpec((R_BLK, 128), lambda r, c: (r, 0)),
            pl.BlockSpec((R_BLK, 128), lambda r, c: (N_RBLK + r, 0)),
        ],
        out_specs=pl.BlockSpec((R_BLK, 128), lambda r, c: (c * N_RBLK + r, 0)),
        out_shape=jax.ShapeDtypeStruct((NC * N_NODES, 128), jnp.float32),
    )(acc1, acc1, y1, y1, b1_2d, w2, degp, degp)


def _pool_body(a0_ref, a1_ref, y0_ref, y1_ref, b_ref, batch_ref,
               p0_ref, p1_ref, w3_ref, b3_ref, out_ref, sums, cnt):
    r = pl.program_id(0)

    @pl.when(r == 0)
    def _init():
        sums[...] = jnp.zeros_like(sums)
        cnt[...] = jnp.zeros_like(cnt)

    deg = p0_ref[:, 0:1] + p1_ref[:, 0:1] + 1.0
    dis = lax.rsqrt(deg)
    h_l = jnp.maximum(dis * (a0_ref[...] + y0_ref[...]) + b_ref[0:1, :], 0.0)
    h_r = jnp.maximum(dis * (a1_ref[...] + y1_ref[...]) + b_ref[1:2, :], 0.0)
    h = jnp.concatenate([h_l, h_r], axis=1)

    gids = lax.broadcasted_iota(jnp.int32, (R_BLK, N_GRAPHS), 1)
    onehot = (batch_ref[...] == gids).astype(jnp.float32)
    sums[...] += lax.dot_general(onehot, h, (((0,), (0,)), ((), ())),
                                 preferred_element_type=jnp.float32)
    cnt[...] += lax.dot_general(onehot, jnp.ones((R_BLK, 8), jnp.float32),
                                (((0,), (0,)), ((), ())),
                                preferred_element_type=jnp.float32)

    @pl.when(r == N_RBLK - 1)
    def _fin():
        g = sums[...] / jnp.maximum(cnt[:, 0:1], 1.0)
        out_ref[...] = jnp.dot(g, w3_ref[...],
                               preferred_element_type=jnp.float32) + b3_ref[...]


def _tc_pool(acc2, y2, b2_2d, batch_2d, degp, w3, b3_2d):
    half = pl.BlockSpec((R_BLK, 128), lambda r: (r, 0))
    half_hi = pl.BlockSpec((R_BLK, 128), lambda r: (N_RBLK + r, 0))
    return pl.pallas_call(
        _pool_body,
        grid=(N_RBLK,),
        in_specs=[
            half, half_hi, half, half_hi,
            pl.BlockSpec((2, 128), lambda r: (0, 0)),
            pl.BlockSpec((R_BLK, 1), lambda r: (r, 0)),
            pl.BlockSpec((R_BLK, 128), lambda r: (r, 0)),
            pl.BlockSpec((R_BLK, 128), lambda r: (N_RBLK + r, 0)),
            pl.BlockSpec((D_HID, D_OUT), lambda r: (0, 0)),
            pl.BlockSpec((1, D_OUT), lambda r: (0, 0)),
        ],
        out_specs=pl.BlockSpec((N_GRAPHS, D_OUT), lambda r: (0, 0)),
        out_shape=jax.ShapeDtypeStruct((N_GRAPHS, D_OUT), jnp.float32),
        scratch_shapes=[
            pltpu.VMEM((N_GRAPHS, D_HID), jnp.float32),
            pltpu.VMEM((N_GRAPHS, 8), jnp.float32),
        ],
        compiler_params=pltpu.CompilerParams(
            dimension_semantics=("arbitrary",)),
    )(acc2, acc2, y2, y2, b2_2d, batch_2d, degp, degp, w3, b3_2d)


def kernel(x, edge_index, batch, W1, b1, W2, b2, W3, b3):
    src = edge_index[0]
    dst = edge_index[1]
    src2 = jnp.concatenate([src, src + N_NODES])

    degp = _sc_degree(dst)
    y1 = _tc_mm1(x, W1, degp)
    acc1 = _sc_edge_pass(y1, src2, dst)
    y2 = _tc_mm2(acc1, y1, b1.reshape(2, 128), W2, degp)
    acc2 = _sc_edge_pass(y2, src2, dst)
    return _tc_pool(acc2, y2, b2.reshape(2, 128), batch.reshape(N_NODES, 1),
                    degp, W3, b3.reshape(1, D_OUT))


# speedup vs baseline: 13.8663x; 1.8637x over previous
import functools

import jax
import jax.numpy as jnp
from jax import lax
from jax.experimental import pallas as pl
from jax.experimental.pallas import tpu as pltpu
from jax.experimental.pallas import tpu_sc as plsc

N_NODES = 10000
N_EDGES = 160000
D_IN = 256
D_HID = 256
D_OUT = 128
N_GRAPHS = 128

NC = 2
NS = 16

ROW_STRIDE = 624
ROW_COPY = 640
EDGES_PER_TILE = N_EDGES // NS
EDGES_PER_TILE_DEG = N_EDGES // (NC * NS)
CH = 80
CH_DEG = 40

R_BLK = 1000
N_RBLK = N_NODES // R_BLK


def _deg_kernel(dst_hbm, ones_hbm, zeros_hbm, out_hbm, dst_v, ones_v, acc, sem):
    c = lax.axis_index("c")
    s = lax.axis_index("s")
    r0 = s * ROW_STRIDE
    pltpu.sync_copy(ones_hbm, ones_v)
    pltpu.sync_copy(zeros_hbm, acc.at[pl.ds(r0, ROW_COPY)])
    plsc.subcore_barrier()

    e0 = (c * NS + s) * EDGES_PER_TILE_DEG

    def body(i, carry):
        base = e0 + i * CH_DEG
        pltpu.sync_copy(dst_hbm.at[pl.ds(base, CH_DEG)], dst_v)
        pltpu.sync_copy(ones_v, acc.at[dst_v], add=True)
        return carry

    lax.fori_loop(0, EDGES_PER_TILE_DEG // CH_DEG, body, 0)
    plsc.subcore_barrier()
    pltpu.sync_copy(
        acc.at[pl.ds(r0, ROW_COPY)],
        out_hbm.at[pl.ds(c * N_NODES + r0, ROW_COPY)],
    )


def _sc_degree(dst):
    ones = jnp.ones((CH_DEG, 128), jnp.float32)
    zeros = jnp.zeros((ROW_COPY, 128), jnp.float32)
    mesh = plsc.VectorSubcoreMesh(core_axis_name="c", subcore_axis_name="s")
    k = functools.partial(
        pl.kernel,
        mesh=mesh,
        out_type=jax.ShapeDtypeStruct((NC * N_NODES, 128), jnp.float32),
        scratch_types=[
            pltpu.VMEM((CH_DEG,), jnp.int32),
            pltpu.VMEM((CH_DEG, 128), jnp.float32),
            pltpu.VMEM_SHARED((N_NODES, 128), jnp.float32),
            pltpu.SemaphoreType.DMA,
        ],
    )(_deg_kernel)
    return k(dst, ones, zeros)


NCHUNK = EDGES_PER_TILE // CH


def _edge_kernel(y_hbm, src2_hbm, dst3_hbm, zeros_hbm, out_hbm,
                 src_all, dst_all, rows0, rows1, acc, sem0, sem1):
    c = lax.axis_index("c")
    s = lax.axis_index("s")
    r0 = s * ROW_STRIDE
    pltpu.sync_copy(zeros_hbm, acc.at[pl.ds(r0, ROW_COPY)])
    pltpu.sync_copy(src2_hbm.at[pl.ds(c * N_EDGES + s * EDGES_PER_TILE,
                                      EDGES_PER_TILE)], src_all)
    pltpu.sync_copy(dst3_hbm.at[s], dst_all)
    plsc.subcore_barrier()

    pltpu.async_copy(y_hbm.at[src_all.at[pl.ds(0, CH)]], rows0, sem0)

    def body(j, carry):
        k0 = 2 * j
        pltpu.async_copy(y_hbm.at[src_all.at[pl.ds((k0 + 1) * CH, CH)]],
                         rows1, sem1)
        pltpu.make_async_copy(y_hbm.at[pl.ds(0, CH)], rows0, sem0).wait()
        pltpu.sync_copy(rows0, acc.at[dst_all.at[k0]], add=True)
        pltpu.async_copy(y_hbm.at[src_all.at[pl.ds((k0 + 2) * CH, CH)]],
                         rows0, sem0)
        pltpu.make_async_copy(y_hbm.at[pl.ds(0, CH)], rows1, sem1).wait()
        pltpu.sync_copy(rows1, acc.at[dst_all.at[k0 + 1]], add=True)
        return carry

    lax.fori_loop(0, (NCHUNK - 1) // 2, body, 0)
    pltpu.make_async_copy(y_hbm.at[pl.ds(0, CH)], rows0, sem0).wait()
    pltpu.sync_copy(rows0, acc.at[dst_all.at[NCHUNK - 1]], add=True)

    plsc.subcore_barrier()
    pltpu.sync_copy(
        acc.at[pl.ds(r0, ROW_COPY)],
        out_hbm.at[pl.ds(c * N_NODES + r0, ROW_COPY)],
    )


def _sc_edge_pass(y_split, src2, dst3):
    zeros = jnp.zeros((ROW_COPY, 128), jnp.float32)
    mesh = plsc.VectorSubcoreMesh(core_axis_name="c", subcore_axis_name="s")
    k = functools.partial(
        pl.kernel,
        mesh=mesh,
        out_type=jax.ShapeDtypeStruct((NC * N_NODES, 128), jnp.float32),
        scratch_types=[
            pltpu.VMEM((EDGES_PER_TILE,), jnp.int32),
            pltpu.VMEM((NCHUNK, CH), jnp.int32),
            pltpu.VMEM((CH, 128), jnp.float32),
            pltpu.VMEM((CH, 128), jnp.float32),
            pltpu.VMEM_SHARED((N_NODES, 128), jnp.float32),
            pltpu.SemaphoreType.DMA,
            pltpu.SemaphoreType.DMA,
        ],
    )(_edge_kernel)
    return k(y_split, src2, dst3, zeros)


def _mm1_body(x_ref, w_ref, p0_ref, p1_ref, out_ref):
    deg = p0_ref[:, 0:1] + p1_ref[:, 0:1] + 1.0
    dis = lax.rsqrt(deg)
    out_ref[...] = dis * jnp.dot(x_ref[...], w_ref[...],
                                 preferred_element_type=jnp.float32)


def _tc_mm1(x, w1, degp):
    return pl.pallas_call(
        _mm1_body,
        grid=(N_RBLK, 2),
        in_specs=[
            pl.BlockSpec((R_BLK, D_IN), lambda r, c: (r, 0)),
            pl.BlockSpec((D_IN, 128), lambda r, c: (0, c)),
            pl.BlockSpec((R_BLK, 128), lambda r, c: (r, 0)),
            pl.BlockSpec((R_BLK, 128), lambda r, c: (N_RBLK + r, 0)),
        ],
        out_specs=pl.BlockSpec((R_BLK, 128), lambda r, c: (c * N_RBLK + r, 0)),
        out_shape=jax.ShapeDtypeStruct((NC * N_NODES, 128), jnp.float32),
    )(x, w1, degp, degp)


def _mm2_body(a0_ref, a1_ref, y0_ref, y1_ref, b_ref, w_ref, p0_ref, p1_ref,
              out_ref):
    deg = p0_ref[:, 0:1] + p1_ref[:, 0:1] + 1.0
    dis = lax.rsqrt(deg)
    h_l = jnp.maximum(dis * (a0_ref[...] + y0_ref[...]) + b_ref[0:1, :], 0.0)
    h_r = jnp.maximum(dis * (a1_ref[...] + y1_ref[...]) + b_ref[1:2, :], 0.0)
    h = jnp.concatenate([h_l, h_r], axis=1)
    out_ref[...] = dis * jnp.dot(h, w_ref[...],
                                 preferred_element_type=jnp.float32)


def _tc_mm2(acc1, y1, b1_2d, w2, degp):
    half = pl.BlockSpec((R_BLK, 128), lambda r, c: (r, 0))
    half_hi = pl.BlockSpec((R_BLK, 128), lambda r, c: (N_RBLK + r, 0))
    return pl.pallas_call(
        _mm2_body,
        grid=(N_RBLK, 2),
        in_specs=[
            half, half_hi, half, half_hi,
            pl.BlockSpec((2, 128), lambda r, c: (0, 0)),
            pl.BlockSpec((D_HID, 128), lambda r, c: (0, c)),
            pl.BlockSpec((R_BLK, 128), lambda r, c: (r, 0)),
            pl.BlockSpec((R_BLK, 128), lambda r, c: (N_RBLK + r, 0)),
        ],
        out_specs=pl.BlockSpec((R_BLK, 128), lambda r, c: (c * N_RBLK + r, 0)),
        out_shape=jax.ShapeDtypeStruct((NC * N_NODES, 128), jnp.float32),
    )(acc1, acc1, y1, y1, b1_2d, w2, degp, degp)


def _pool_body(a0_ref, a1_ref, y0_ref, y1_ref, b_ref, batch_ref,
               p0_ref, p1_ref, w3_ref, b3_ref, out_ref, sums, cnt):
    r = pl.program_id(0)

    @pl.when(r == 0)
    def _init():
        sums[...] = jnp.zeros_like(sums)
        cnt[...] = jnp.zeros_like(cnt)

    deg = p0_ref[:, 0:1] + p1_ref[:, 0:1] + 1.0
    dis = lax.rsqrt(deg)
    h_l = jnp.maximum(dis * (a0_ref[...] + y0_ref[...]) + b_ref[0:1, :], 0.0)
    h_r = jnp.maximum(dis * (a1_ref[...] + y1_ref[...]) + b_ref[1:2, :], 0.0)
    h = jnp.concatenate([h_l, h_r], axis=1)

    gids = lax.broadcasted_iota(jnp.int32, (R_BLK, N_GRAPHS), 1)
    onehot = (batch_ref[...] == gids).astype(jnp.float32)
    sums[...] += lax.dot_general(onehot, h, (((0,), (0,)), ((), ())),
                                 preferred_element_type=jnp.float32)
    cnt[...] += lax.dot_general(onehot, jnp.ones((R_BLK, 8), jnp.float32),
                                (((0,), (0,)), ((), ())),
                                preferred_element_type=jnp.float32)

    @pl.when(r == N_RBLK - 1)
    def _fin():
        g = sums[...] / jnp.maximum(cnt[:, 0:1], 1.0)
        out_ref[...] = jnp.dot(g, w3_ref[...],
                               preferred_element_type=jnp.float32) + b3_ref[...]


def _tc_pool(acc2, y2, b2_2d, batch_2d, degp, w3, b3_2d):
    half = pl.BlockSpec((R_BLK, 128), lambda r: (r, 0))
    half_hi = pl.BlockSpec((R_BLK, 128), lambda r: (N_RBLK + r, 0))
    return pl.pallas_call(
        _pool_body,
        grid=(N_RBLK,),
        in_specs=[
            half, half_hi, half, half_hi,
            pl.BlockSpec((2, 128), lambda r: (0, 0)),
            pl.BlockSpec((R_BLK, 1), lambda r: (r, 0)),
            pl.BlockSpec((R_BLK, 128), lambda r: (r, 0)),
            pl.BlockSpec((R_BLK, 128), lambda r: (N_RBLK + r, 0)),
            pl.BlockSpec((D_HID, D_OUT), lambda r: (0, 0)),
            pl.BlockSpec((1, D_OUT), lambda r: (0, 0)),
        ],
        out_specs=pl.BlockSpec((N_GRAPHS, D_OUT), lambda r: (0, 0)),
        out_shape=jax.ShapeDtypeStruct((N_GRAPHS, D_OUT), jnp.float32),
        scratch_shapes=[
            pltpu.VMEM((N_GRAPHS, D_HID), jnp.float32),
            pltpu.VMEM((N_GRAPHS, 8), jnp.float32),
        ],
        compiler_params=pltpu.CompilerParams(
            dimension_semantics=("arbitrary",)),
    )(acc2, acc2, y2, y2, b2_2d, batch_2d, degp, degp, w3, b3_2d)


def kernel(x, edge_index, batch, W1, b1, W2, b2, W3, b3):
    src = edge_index[0]
    dst = edge_index[1]
    src2 = jnp.concatenate([src, src + N_NODES])
    dst3 = dst.reshape(NS, NCHUNK, CH)

    degp = _sc_degree(dst)
    y1 = _tc_mm1(x, W1, degp)
    acc1 = _sc_edge_pass(y1, src2, dst3)
    y2 = _tc_mm2(acc1, y1, b1.reshape(2, 128), W2, degp)
    acc2 = _sc_edge_pass(y2, src2, dst3)
    return _tc_pool(acc2, y2, b2.reshape(2, 128), batch.reshape(N_NODES, 1),
                    degp, W3, b3.reshape(1, D_OUT))


# speedup vs baseline: 15.8914x; 1.1460x over previous
import functools

import jax
import jax.numpy as jnp
from jax import lax
from jax.experimental import pallas as pl
from jax.experimental.pallas import tpu as pltpu
from jax.experimental.pallas import tpu_sc as plsc

N_NODES = 10000
N_EDGES = 160000
D_IN = 256
D_HID = 256
D_OUT = 128
N_GRAPHS = 128

NC = 2
NS = 16

ROW_STRIDE = 624
ROW_COPY = 640
EDGES_PER_TILE = N_EDGES // NS
EDGES_PER_TILE_DEG = N_EDGES // (NC * NS)
CH = 80
CH_DEG = 40

R_BLK = 1000
N_RBLK = N_NODES // R_BLK


NCHUNK_DEG = EDGES_PER_TILE_DEG // CH_DEG


def _deg_kernel(dst3_hbm, ones_hbm, zeros_hbm, out_hbm, dst_all, ones_v, acc,
                sem):
    c = lax.axis_index("c")
    s = lax.axis_index("s")
    r0 = s * ROW_STRIDE
    pltpu.sync_copy(ones_hbm, ones_v)
    pltpu.sync_copy(zeros_hbm, acc.at[pl.ds(r0, ROW_COPY)])
    pltpu.sync_copy(dst3_hbm.at[c * NS + s], dst_all)
    plsc.subcore_barrier()

    def body(k, carry):
        pltpu.async_copy(ones_v, acc.at[dst_all.at[k]], sem, add=True)

        @pl.when(k >= 4)
        def _w():
            pltpu.make_async_copy(ones_hbm, ones_v, sem).wait()

        return carry

    lax.fori_loop(0, NCHUNK_DEG, body, 0)
    for _ in range(4):
        pltpu.make_async_copy(ones_hbm, ones_v, sem).wait()
    plsc.subcore_barrier()
    pltpu.sync_copy(
        acc.at[pl.ds(r0, ROW_COPY)],
        out_hbm.at[pl.ds(c * N_NODES + r0, ROW_COPY)],
    )


def _sc_degree(dst):
    ones = jnp.ones((CH_DEG, 128), jnp.float32)
    zeros = jnp.zeros((ROW_COPY, 128), jnp.float32)
    dst3 = dst.reshape(NC * NS, NCHUNK_DEG, CH_DEG)
    mesh = plsc.VectorSubcoreMesh(core_axis_name="c", subcore_axis_name="s")
    k = functools.partial(
        pl.kernel,
        mesh=mesh,
        out_type=jax.ShapeDtypeStruct((NC * N_NODES, 128), jnp.float32),
        scratch_types=[
            pltpu.VMEM((NCHUNK_DEG, CH_DEG), jnp.int32),
            pltpu.VMEM((CH_DEG, 128), jnp.float32),
            pltpu.VMEM_SHARED((N_NODES, 128), jnp.float32),
            pltpu.SemaphoreType.DMA,
        ],
    )(_deg_kernel)
    return k(dst3, ones, zeros)


NCHUNK = EDGES_PER_TILE // CH


def _edge_kernel(y_hbm, src2_hbm, dst3_hbm, zeros_hbm, out_hbm,
                 src_all, dst_all, rows0, rows1, acc, g0, g1, s0, s1):
    rows = [rows0, rows1]
    gsem = [g0, g1]
    ssem = [s0, s1]
    c = lax.axis_index("c")
    s = lax.axis_index("s")
    r0 = s * ROW_STRIDE
    pltpu.sync_copy(zeros_hbm, acc.at[pl.ds(r0, ROW_COPY)])
    pltpu.sync_copy(src2_hbm.at[pl.ds(c * N_EDGES + s * EDGES_PER_TILE,
                                      EDGES_PER_TILE)], src_all)
    pltpu.sync_copy(dst3_hbm.at[s], dst_all)
    plsc.subcore_barrier()

    pltpu.async_copy(y_hbm.at[src_all.at[pl.ds(0, CH)]], rows0, g0)
    pltpu.async_copy(y_hbm.at[src_all.at[pl.ds(CH, CH)]], rows1, g1)

    def body(j, carry):
        for b in range(2):
            k = 2 * j + b
            pltpu.make_async_copy(y_hbm.at[pl.ds(0, CH)], rows[b],
                                  gsem[b]).wait()
            pltpu.async_copy(rows[b], acc.at[dst_all.at[k]], ssem[b],
                             add=True)

            @pl.when(k + 2 <= NCHUNK - 1)
            def _g():
                pltpu.make_async_copy(y_hbm.at[pl.ds(0, CH)], rows[b],
                                      ssem[b]).wait()
                pltpu.async_copy(
                    y_hbm.at[src_all.at[pl.ds((k + 2) * CH, CH)]],
                    rows[b], gsem[b])

        return carry

    lax.fori_loop(0, (NCHUNK - 1) // 2, body, 0)
    pltpu.make_async_copy(y_hbm.at[pl.ds(0, CH)], rows0, g0).wait()
    pltpu.sync_copy(rows0, acc.at[dst_all.at[NCHUNK - 1]], add=True)
    pltpu.make_async_copy(y_hbm.at[pl.ds(0, CH)], rows1, s1).wait()

    plsc.subcore_barrier()
    pltpu.sync_copy(
        acc.at[pl.ds(r0, ROW_COPY)],
        out_hbm.at[pl.ds(c * N_NODES + r0, ROW_COPY)],
    )


def _sc_edge_pass(y_split, src2, dst3):
    zeros = jnp.zeros((ROW_COPY, 128), jnp.float32)
    mesh = plsc.VectorSubcoreMesh(core_axis_name="c", subcore_axis_name="s")
    k = functools.partial(
        pl.kernel,
        mesh=mesh,
        out_type=jax.ShapeDtypeStruct((NC * N_NODES, 128), jnp.float32),
        scratch_types=[
            pltpu.VMEM((EDGES_PER_TILE,), jnp.int32),
            pltpu.VMEM((NCHUNK, CH), jnp.int32),
            pltpu.VMEM((CH, 128), jnp.float32),
            pltpu.VMEM((CH, 128), jnp.float32),
            pltpu.VMEM_SHARED((N_NODES, 128), jnp.float32),
            pltpu.SemaphoreType.DMA,
            pltpu.SemaphoreType.DMA,
            pltpu.SemaphoreType.DMA,
            pltpu.SemaphoreType.DMA,
        ],
    )(_edge_kernel)
    return k(y_split, src2, dst3, zeros)


def _mm1_body(x_ref, w_ref, p0_ref, p1_ref, out_ref):
    deg = p0_ref[:, 0:1] + p1_ref[:, 0:1] + 1.0
    dis = lax.rsqrt(deg)
    out_ref[...] = dis * jnp.dot(x_ref[...], w_ref[...],
                                 preferred_element_type=jnp.float32)


def _tc_mm1(x, w1, degp):
    return pl.pallas_call(
        _mm1_body,
        grid=(N_RBLK, 2),
        in_specs=[
            pl.BlockSpec((R_BLK, D_IN), lambda r, c: (r, 0)),
            pl.BlockSpec((D_IN, 128), lambda r, c: (0, c)),
            pl.BlockSpec((R_BLK, 128), lambda r, c: (r, 0)),
            pl.BlockSpec((R_BLK, 128), lambda r, c: (N_RBLK + r, 0)),
        ],
        out_specs=pl.BlockSpec((R_BLK, 128), lambda r, c: (c * N_RBLK + r, 0)),
        out_shape=jax.ShapeDtypeStruct((NC * N_NODES, 128), jnp.float32),
    )(x, w1, degp, degp)


def _mm2_body(a0_ref, a1_ref, y0_ref, y1_ref, b_ref, w_ref, p0_ref, p1_ref,
              out_ref):
    deg = p0_ref[:, 0:1] + p1_ref[:, 0:1] + 1.0
    dis = lax.rsqrt(deg)
    h_l = jnp.maximum(dis * (a0_ref[...] + y0_ref[...]) + b_ref[0:1, :], 0.0)
    h_r = jnp.maximum(dis * (a1_ref[...] + y1_ref[...]) + b_ref[1:2, :], 0.0)
    h = jnp.concatenate([h_l, h_r], axis=1)
    out_ref[...] = dis * jnp.dot(h, w_ref[...],
                                 preferred_element_type=jnp.float32)


def _tc_mm2(acc1, y1, b1_2d, w2, degp):
    half = pl.BlockSpec((R_BLK, 128), lambda r, c: (r, 0))
    half_hi = pl.BlockSpec((R_BLK, 128), lambda r, c: (N_RBLK + r, 0))
    return pl.pallas_call(
        _mm2_body,
        grid=(N_RBLK, 2),
        in_specs=[
            half, half_hi, half, half_hi,
            pl.BlockSpec((2, 128), lambda r, c: (0, 0)),
            pl.BlockSpec((D_HID, 128), lambda r, c: (0, c)),
            pl.BlockSpec((R_BLK, 128), lambda r, c: (r, 0)),
            pl.BlockSpec((R_BLK, 128), lambda r, c: (N_RBLK + r, 0)),
        ],
        out_specs=pl.BlockSpec((R_BLK, 128), lambda r, c: (c * N_RBLK + r, 0)),
        out_shape=jax.ShapeDtypeStruct((NC * N_NODES, 128), jnp.float32),
    )(acc1, acc1, y1, y1, b1_2d, w2, degp, degp)


def _pool_body(a0_ref, a1_ref, y0_ref, y1_ref, b_ref, batch_ref,
               p0_ref, p1_ref, w3_ref, b3_ref, out_ref, sums, cnt):
    r = pl.program_id(0)

    @pl.when(r == 0)
    def _init():
        sums[...] = jnp.zeros_like(sums)
        cnt[...] = jnp.zeros_like(cnt)

    deg = p0_ref[:, 0:1] + p1_ref[:, 0:1] + 1.0
    dis = lax.rsqrt(deg)
    h_l = jnp.maximum(dis * (a0_ref[...] + y0_ref[...]) + b_ref[0:1, :], 0.0)
    h_r = jnp.maximum(dis * (a1_ref[...] + y1_ref[...]) + b_ref[1:2, :], 0.0)
    h = jnp.concatenate([h_l, h_r], axis=1)

    gids = lax.broadcasted_iota(jnp.int32, (R_BLK, N_GRAPHS), 1)
    onehot = (batch_ref[...] == gids).astype(jnp.float32)
    sums[...] += lax.dot_general(onehot, h, (((0,), (0,)), ((), ())),
                                 preferred_element_type=jnp.float32)
    cnt[...] += lax.dot_general(onehot, jnp.ones((R_BLK, 8), jnp.float32),
                                (((0,), (0,)), ((), ())),
                                preferred_element_type=jnp.float32)

    @pl.when(r == N_RBLK - 1)
    def _fin():
        g = sums[...] / jnp.maximum(cnt[:, 0:1], 1.0)
        out_ref[...] = jnp.dot(g, w3_ref[...],
                               preferred_element_type=jnp.float32) + b3_ref[...]


def _tc_pool(acc2, y2, b2_2d, batch_2d, degp, w3, b3_2d):
    half = pl.BlockSpec((R_BLK, 128), lambda r: (r, 0))
    half_hi = pl.BlockSpec((R_BLK, 128), lambda r: (N_RBLK + r, 0))
    return pl.pallas_call(
        _pool_body,
        grid=(N_RBLK,),
        in_specs=[
            half, half_hi, half, half_hi,
            pl.BlockSpec((2, 128), lambda r: (0, 0)),
            pl.BlockSpec((R_BLK, 1), lambda r: (r, 0)),
            pl.BlockSpec((R_BLK, 128), lambda r: (r, 0)),
            pl.BlockSpec((R_BLK, 128), lambda r: (N_RBLK + r, 0)),
            pl.BlockSpec((D_HID, D_OUT), lambda r: (0, 0)),
            pl.BlockSpec((1, D_OUT), lambda r: (0, 0)),
        ],
        out_specs=pl.BlockSpec((N_GRAPHS, D_OUT), lambda r: (0, 0)),
        out_shape=jax.ShapeDtypeStruct((N_GRAPHS, D_OUT), jnp.float32),
        scratch_shapes=[
            pltpu.VMEM((N_GRAPHS, D_HID), jnp.float32),
            pltpu.VMEM((N_GRAPHS, 8), jnp.float32),
        ],
        compiler_params=pltpu.CompilerParams(
            dimension_semantics=("arbitrary",)),
    )(acc2, acc2, y2, y2, b2_2d, batch_2d, degp, degp, w3, b3_2d)


def kernel(x, edge_index, batch, W1, b1, W2, b2, W3, b3):
    src = edge_index[0]
    dst = edge_index[1]
    src2 = jnp.concatenate([src, src + N_NODES])
    dst3 = dst.reshape(NS, NCHUNK, CH)

    degp = _sc_degree(dst)
    y1 = _tc_mm1(x, W1, degp)
    acc1 = _sc_edge_pass(y1, src2, dst3)
    y2 = _tc_mm2(acc1, y1, b1.reshape(2, 128), W2, degp)
    acc2 = _sc_edge_pass(y2, src2, dst3)
    return _tc_pool(acc2, y2, b2.reshape(2, 128), batch.reshape(N_NODES, 1),
                    degp, W3, b3.reshape(1, D_OUT))
